# Initial kernel scaffold; baseline (speedup 1.0000x reference)
import jax
import jax.numpy as jnp
from jax.experimental import pallas as pl


def kernel(input_y):
    raise NotImplementedError("write your pallas kernel here")


# speedup vs baseline: 4.5809x; 4.5809x over previous
import jax
import jax.numpy as jnp
from jax.experimental import pallas as pl
from jax.experimental.pallas import tpu as pltpu

P = 11
KSEL = 16
V = 32
LH = 118
NH = 40
NPOS = 1600
NPOSP = 1664
NROW = 65 * 9 * 8


def _dist_kernel(x_ref, xsh_ref, nq_ref, d_ref, c1cap, trb, c2t, c2b, stage):
    dri = pl.program_id(1)
    dr = dri - 32

    zero8 = jnp.zeros((8, 128), jnp.float32)

    for g in range(9):
        c1cap[8 * g:8 * g + 8, 40, :] = zero8

    def r_body(r, accs):
        xrow = x_ref[0, pl.ds(r, 1), :]
        bx = jnp.broadcast_to(xrow, (8, 128))
        new = []
        for g in range(9):
            xs = xsh_ref[0, g, :, pl.ds(dri + r, 1), :]
            new.append(accs[g] + bx * xs.reshape(8, 128))
        new = tuple(new)

        top = (r >= 10) & (jax.lax.rem(r - 10, 3) == 0)
        bot = (r >= 2) & (r <= 116) & (jax.lax.rem(r - 2, 3) == 0)

        @pl.when(top)
        def _():
            slot = (r - 10) // 3
            for g in range(9):
                c1cap[8 * g:8 * g + 8, pl.ds(slot, 1), :] = new[g][:, None, :]

        @pl.when(bot)
        def _():
            slot = 40 + (r + 1) // 3
            for g in range(9):
                c1cap[8 * g:8 * g + 8, pl.ds(slot, 1), :] = new[g][:, None, :]

        return new

    jax.lax.fori_loop(0, 128, r_body, tuple(zero8 for _ in range(9)))

    lane_i = jax.lax.broadcasted_iota(jnp.int32, (8, 128), 1)
    for g in range(9):
        for dcl in range(8):
            slab = c1cap[8 * g + dcl]
            trb[dcl, :, 0:80] = jnp.transpose(slab)

        c2b[0] = zero8

        def w_body(w, acc2):
            tile = trb[:, pl.ds(w, 1), :].reshape(8, 128)
            acc2 = acc2 + tile

            @pl.when((w >= 10) & (jax.lax.rem(w - 10, 3) == 0))
            def _():
                c2t[pl.ds((w - 10) // 3, 1)] = acc2[None]

            @pl.when((w >= 2) & (w <= 116) & (jax.lax.rem(w - 2, 3) == 0))
            def _():
                c2b[pl.ds((w + 1) // 3, 1)] = acc2[None]

            return acc2

        jax.lax.fori_loop(0, 128, w_body, zero8)

        dci_sub = 8 * g + jax.lax.broadcasted_iota(jnp.int32, (8, 128), 0)
        rowpos = 3 * lane_i + dr
        for wj in range(40):
            a = c2t[wj]
            b2 = c2b[wj]
            e1 = a - pltpu.roll(a, 88, axis=1)
            e2 = e1 - b2
            box = e2 + pltpu.roll(b2, 88, axis=1)
            ncand = nq_ref[0, 0, pl.ds(3 * wj + 8 * g, 8), :]
            d = (ncand - 2.0 * box) / 121.0
            colpos = 3 * wj + dci_sub - 32
            valid = ((rowpos >= 0) & (rowpos <= 117) & (colpos >= 0)
                     & (colpos <= 117) & (dci_sub <= 64) & (lane_i < 40))
            d = jnp.where(valid, d, jnp.inf)
            d = jnp.where((dri == 32) & (dci_sub == 32), -jnp.inf, d)
            s = (40 * wj) % 128
            b0 = (40 * wj) // 128
            rolled = pltpu.roll(d, s, axis=1)
            m0 = (lane_i >= s) & (lane_i < min(s + 40, 128))
            stage[:, 128 * b0:128 * b0 + 128] = jnp.where(
                m0, rolled, stage[:, 128 * b0:128 * b0 + 128])
            if s + 40 > 128:
                m1 = lane_i < (s + 40 - 128)
                stage[:, 128 * (b0 + 1):128 * (b0 + 2)] = jnp.where(
                    m1, rolled, stage[:, 128 * (b0 + 1):128 * (b0 + 2)])
        d_ref[0, 0, g] = stage[:, :]


def _topk_kernel(d_ref, gidx_ref, vs, enc):
    pb = pl.program_id(1)
    vs[:, :] = d_ref[0].reshape(NROW, 128)
    enc[:, :] = jax.lax.broadcasted_iota(jnp.int32, (NROW, 128), 0)
    lane = jax.lax.broadcasted_iota(jnp.int32, (1, 128), 1)
    posp = pb * 128 + lane
    wj = posp // 40
    hi = posp - 40 * wj

    def k_body(k, _):
        v = vs[:, :]
        e = enc[:, :]
        m = jnp.min(v, axis=0, keepdims=True)
        ii = jnp.min(jnp.where(v == m, e, jnp.int32(0x3FFFFFFF)),
                     axis=0, keepdims=True)
        vs[:, :] = jnp.where(e == ii, jnp.inf, v)
        dri = ii // 72
        dci = ii - 72 * dri
        row = 3 * hi + dri - 32
        col = 3 * wj + dci - 32
        gidx_ref[0, pl.ds(k, 1), :] = row * LH + col
        return 0

    jax.lax.fori_loop(0, KSEL, k_body, 0)


def _gather_kernel(x_ref, gidx_ref, y_ref, smem, sem):
    copy = pltpu.make_async_copy(gidx_ref.at[0], smem, sem)
    copy.start()
    copy.wait()
    for li in range(128):
        for k in range(KSEL):
            g = smem[k, li]
            gi = g // LH
            gj = g - LH * gi
            rows = x_ref[0, pl.ds(gi, P), :]
            rolled = pltpu.roll(rows, jax.lax.rem(128 - gj, 128), axis=1)
            y_ref[0, li, k, :, :] = rolled[:, 0:P]


def _box_sum_host(img, p):
    c = jnp.cumsum(jnp.cumsum(img, axis=1), axis=2)
    c = jnp.pad(c, ((0, 0), (1, 0), (1, 0)))
    return c[:, p:, p:] - c[:, :-p, p:] - c[:, p:, :-p] + c[:, :-p, :-p]


@jax.jit
def kernel(input_y):
    x = input_y[:, 0]
    n = x.shape[0]

    xpad = jnp.pad(x, ((0, 0), (V, V), (V, 128 - V)))
    xsh = jnp.stack([xpad[:, :, s:s + 128] for s in range(72)], axis=1)
    xsh = xsh.reshape(n, 9, 8, 192, 128)

    norm = _box_sum_host(x * x, P)
    normep = jnp.pad(norm, ((0, 0), (V, V), (V, V)), mode="edge")
    rows = 3 * jnp.arange(NH)[None, :] + jnp.arange(65)[:, None]
    nq = jnp.transpose(normep[:, rows, :], (0, 1, 3, 2))
    nq = jnp.pad(nq, ((0, 0), (0, 0), (0, 10), (0, 88)))

    dists = pl.pallas_call(
        _dist_kernel,
        grid=(n, 65),
        in_specs=[
            pl.BlockSpec((1, 128, 128), lambda i, j: (i, 0, 0)),
            pl.BlockSpec((1, 9, 8, 192, 128), lambda i, j: (i, 0, 0, 0, 0)),
            pl.BlockSpec((1, 1, 192, 128), lambda i, j: (i, j, 0, 0)),
        ],
        out_specs=pl.BlockSpec((1, 1, 9, 8, NPOSP), lambda i, j: (i, j, 0, 0, 0)),
        out_shape=jax.ShapeDtypeStruct((n, 65, 9, 8, NPOSP), jnp.float32),
        scratch_shapes=[
            pltpu.VMEM((72, 80, 128), jnp.float32),
            pltpu.VMEM((8, 128, 128), jnp.float32),
            pltpu.VMEM((40, 8, 128), jnp.float32),
            pltpu.VMEM((40, 8, 128), jnp.float32),
            pltpu.VMEM((8, NPOSP), jnp.float32),
        ],
        compiler_params=pltpu.CompilerParams(
            dimension_semantics=("parallel", "arbitrary"),
        ),
    )(x, xsh, nq)

    gidx = pl.pallas_call(
        _topk_kernel,
        grid=(n, NPOSP // 128),
        in_specs=[pl.BlockSpec((1, 65, 9, 8, 128), lambda i, j: (i, 0, 0, 0, j))],
        out_specs=pl.BlockSpec((1, KSEL, 128), lambda i, j: (i, 0, j)),
        out_shape=jax.ShapeDtypeStruct((n, KSEL, NPOSP), jnp.int32),
        scratch_shapes=[
            pltpu.VMEM((NROW, 128), jnp.float32),
            pltpu.VMEM((NROW, 128), jnp.int32),
        ],
        compiler_params=pltpu.CompilerParams(
            dimension_semantics=("parallel", "arbitrary"),
        ),
    )(dists)

    y5 = pl.pallas_call(
        _gather_kernel,
        grid=(n, NPOSP // 128),
        in_specs=[
            pl.BlockSpec((1, 128, 128), lambda i, j: (i, 0, 0)),
            pl.BlockSpec((1, KSEL, 128), lambda i, j: (i, 0, j)),
        ],
        out_specs=pl.BlockSpec((1, 128, KSEL, P, P), lambda i, j: (i, j, 0, 0, 0)),
        out_shape=jax.ShapeDtypeStruct((n, NPOSP, KSEL, P, P), jnp.float32),
        scratch_shapes=[
            pltpu.SMEM((KSEL, 128), jnp.int32),
            pltpu.SemaphoreType.DMA,
        ],
        compiler_params=pltpu.CompilerParams(
            dimension_semantics=("parallel", "arbitrary"),
        ),
    )(x, gidx)

    y = y5[:, :NPOS].reshape(n, NH, NH, KSEL, P * P)
    return jnp.transpose(y, (0, 2, 1, 3, 4)).reshape(n, NPOS, KSEL, P * P)


# speedup vs baseline: 17.3759x; 3.7932x over previous
import jax
import jax.numpy as jnp
from jax.experimental import pallas as pl
from jax.experimental.pallas import tpu as pltpu

P = 11
KSEL = 16
V = 32
LH = 118
NH = 40
NPOS = 1600
NPOSP = 1664
NROW = 65 * 9 * 8


def _dist_kernel(x_ref, xsh_ref, nq_ref, d_ref, c1cap, trb, c2t, c2b, stage):
    dri = pl.program_id(1)
    dr = dri - 32

    zero72 = jnp.zeros((72, 128), jnp.float32)

    c1cap[:, 40, :] = zero72

    acc = zero72
    for r in range(128):
        xrow = x_ref[0, pl.ds(r, 1), :]
        bx = jnp.broadcast_to(xrow, (72, 128))
        xs = xsh_ref[0, pl.ds(dri + r, 1), :, :].reshape(72, 128)
        acc = acc + bx * xs
        if r >= 10 and (r - 10) % 3 == 0:
            c1cap[:, (r - 10) // 3, :] = acc
        if 2 <= r <= 116 and (r - 2) % 3 == 0:
            c1cap[:, 40 + (r + 1) // 3, :] = acc

    for dc in range(72):
        trb[dc, :, 0:80] = jnp.transpose(c1cap[dc])

    c2b[0] = zero72
    acc2 = zero72
    for w in range(128):
        acc2 = acc2 + trb[:, pl.ds(w, 1), :].reshape(72, 128)
        if w >= 10 and (w - 10) % 3 == 0:
            c2t[(w - 10) // 3] = acc2
        if 2 <= w <= 116 and (w - 2) % 3 == 0:
            c2b[(w + 1) // 3] = acc2

    lane_i = jax.lax.broadcasted_iota(jnp.int32, (72, 128), 1)
    dci_sub = jax.lax.broadcasted_iota(jnp.int32, (72, 128), 0)
    rowpos = 3 * lane_i + dr
    for wj in range(40):
        a = c2t[wj]
        b2 = c2b[wj]
        e1 = a - pltpu.roll(a, 88, axis=1)
        e2 = e1 - b2
        box = e2 + pltpu.roll(b2, 88, axis=1)
        ncand = nq_ref[0, 0, 3 * wj:3 * wj + 72, :]
        d = (ncand - 2.0 * box) / 121.0
        colpos = 3 * wj + dci_sub - 32
        valid = ((rowpos >= 0) & (rowpos <= 117) & (colpos >= 0)
                 & (colpos <= 117) & (dci_sub <= 64) & (lane_i < 40))
        d = jnp.where(valid, d, jnp.inf)
        d = jnp.where((dri == 32) & (dci_sub == 32), -jnp.inf, d)
        s = (40 * wj) % 128
        b0 = (40 * wj) // 128
        rolled = pltpu.roll(d, s, axis=1)
        m0 = (lane_i >= s) & (lane_i < min(s + 40, 128))
        stage[:, 128 * b0:128 * b0 + 128] = jnp.where(
            m0, rolled, stage[:, 128 * b0:128 * b0 + 128])
        if s + 40 > 128:
            m1 = lane_i < (s + 40 - 128)
            stage[:, 128 * (b0 + 1):128 * (b0 + 2)] = jnp.where(
                m1, rolled, stage[:, 128 * (b0 + 1):128 * (b0 + 2)])
    d_ref[0, 0] = stage[:, :].reshape(9, 8, NPOSP)


def _topk_kernel(d_ref, gidx_ref, vs, enc):
    pb = pl.program_id(1)
    vs[:, :] = d_ref[0].reshape(NROW, 128)
    enc[:, :] = jax.lax.broadcasted_iota(jnp.int32, (NROW, 128), 0)
    lane = jax.lax.broadcasted_iota(jnp.int32, (1, 128), 1)
    posp = pb * 128 + lane
    wj = posp // 40
    hi = posp - 40 * wj

    def k_body(k, _):
        v = vs[:, :]
        e = enc[:, :]
        m = jnp.min(v, axis=0, keepdims=True)
        ii = jnp.min(jnp.where(v == m, e, jnp.int32(0x3FFFFFFF)),
                     axis=0, keepdims=True)
        vs[:, :] = jnp.where(e == ii, jnp.inf, v)
        dri = ii // 72
        dci = ii - 72 * dri
        row = 3 * hi + dri - 32
        col = 3 * wj + dci - 32
        gidx_ref[0, pl.ds(k, 1), :] = row * LH + col
        return 0

    jax.lax.fori_loop(0, KSEL, k_body, 0)


def _gather_kernel(x_ref, gidx_ref, y_ref, smem, sem):
    copy = pltpu.make_async_copy(gidx_ref.at[0], smem, sem)
    copy.start()
    copy.wait()
    for li in range(128):
        for k in range(KSEL):
            g = smem[k, li]
            gi = g // LH
            gj = g - LH * gi
            rows = x_ref[0, pl.ds(gi, P), :]
            rolled = pltpu.roll(rows, jax.lax.rem(128 - gj, 128), axis=1)
            y_ref[0, li, k, :, :] = rolled[:, 0:P]


def _box_sum_host(img, p):
    c = jnp.cumsum(jnp.cumsum(img, axis=1), axis=2)
    c = jnp.pad(c, ((0, 0), (1, 0), (1, 0)))
    return c[:, p:, p:] - c[:, :-p, p:] - c[:, p:, :-p] + c[:, :-p, :-p]


@jax.jit
def kernel(input_y):
    x = input_y[:, 0]
    n = x.shape[0]

    xpad = jnp.pad(x, ((0, 0), (V, V), (V, 128 - V)))
    xsh = jnp.stack([xpad[:, :, s:s + 128] for s in range(72)], axis=2)

    norm = _box_sum_host(x * x, P)
    normep = jnp.pad(norm, ((0, 0), (V, V), (V, V)), mode="edge")
    rows = 3 * jnp.arange(NH)[None, :] + jnp.arange(65)[:, None]
    nq = jnp.transpose(normep[:, rows, :], (0, 1, 3, 2))
    nq = jnp.pad(nq, ((0, 0), (0, 0), (0, 10), (0, 88)))

    dists = pl.pallas_call(
        _dist_kernel,
        grid=(n, 65),
        in_specs=[
            pl.BlockSpec((1, 128, 128), lambda i, j: (i, 0, 0)),
            pl.BlockSpec((1, 192, 72, 128), lambda i, j: (i, 0, 0, 0)),
            pl.BlockSpec((1, 1, 192, 128), lambda i, j: (i, j, 0, 0)),
        ],
        out_specs=pl.BlockSpec((1, 1, 9, 8, NPOSP), lambda i, j: (i, j, 0, 0, 0)),
        out_shape=jax.ShapeDtypeStruct((n, 65, 9, 8, NPOSP), jnp.float32),
        scratch_shapes=[
            pltpu.VMEM((72, 80, 128), jnp.float32),
            pltpu.VMEM((72, 128, 128), jnp.float32),
            pltpu.VMEM((40, 72, 128), jnp.float32),
            pltpu.VMEM((40, 72, 128), jnp.float32),
            pltpu.VMEM((72, NPOSP), jnp.float32),
        ],
        compiler_params=pltpu.CompilerParams(
            dimension_semantics=("parallel", "arbitrary"),
        ),
    )(x, xsh, nq)

    gidx = pl.pallas_call(
        _topk_kernel,
        grid=(n, NPOSP // 128),
        in_specs=[pl.BlockSpec((1, 65, 9, 8, 128), lambda i, j: (i, 0, 0, 0, j))],
        out_specs=pl.BlockSpec((1, KSEL, 128), lambda i, j: (i, 0, j)),
        out_shape=jax.ShapeDtypeStruct((n, KSEL, NPOSP), jnp.int32),
        scratch_shapes=[
            pltpu.VMEM((NROW, 128), jnp.float32),
            pltpu.VMEM((NROW, 128), jnp.int32),
        ],
        compiler_params=pltpu.CompilerParams(
            dimension_semantics=("parallel", "arbitrary"),
        ),
    )(dists)

    y5 = pl.pallas_call(
        _gather_kernel,
        grid=(n, NPOSP // 128),
        in_specs=[
            pl.BlockSpec((1, 128, 128), lambda i, j: (i, 0, 0)),
            pl.BlockSpec((1, KSEL, 128), lambda i, j: (i, 0, j)),
        ],
        out_specs=pl.BlockSpec((1, 128, KSEL, P, P), lambda i, j: (i, j, 0, 0, 0)),
        out_shape=jax.ShapeDtypeStruct((n, NPOSP, KSEL, P, P), jnp.float32),
        scratch_shapes=[
            pltpu.SMEM((KSEL, 128), jnp.int32),
            pltpu.SemaphoreType.DMA,
        ],
        compiler_params=pltpu.CompilerParams(
            dimension_semantics=("parallel", "arbitrary"),
        ),
    )(x, gidx)

    y = y5[:, :NPOS].reshape(n, NH, NH, KSEL, P * P)
    return jnp.transpose(y, (0, 2, 1, 3, 4)).reshape(n, NPOS, KSEL, P * P)


# speedup vs baseline: 19.8325x; 1.1414x over previous
import jax
import jax.numpy as jnp
from jax.experimental import pallas as pl
from jax.experimental.pallas import tpu as pltpu

P = 11
KSEL = 16
V = 32
LH = 118
NH = 40
NPOS = 1600
NPOSP = 1664
NROW = 65 * 9 * 8


def _dist_kernel(x_ref, xsh_ref, nq_ref, d_ref, c1cap, trb, c2t, c2b, stage):
    dri = pl.program_id(1)
    dr = dri - 32

    zero72 = jnp.zeros((72, 128), jnp.float32)

    c1cap[:, 40, :] = zero72

    acc = zero72
    for r in range(128):
        xrow = x_ref[0, pl.ds(r, 1), :]
        bx = jnp.broadcast_to(xrow, (72, 128))
        xs = xsh_ref[0, pl.ds(dri + r, 1), :, :].reshape(72, 128)
        acc = acc + bx * xs
        if r >= 10 and (r - 10) % 3 == 0:
            c1cap[:, (r - 10) // 3, :] = acc
        if 2 <= r <= 116 and (r - 2) % 3 == 0:
            c1cap[:, 40 + (r + 1) // 3, :] = acc

    for dc in range(72):
        trb[dc, :, 0:80] = jnp.transpose(c1cap[dc])

    c2b[0] = zero72
    acc2 = zero72
    for w in range(128):
        acc2 = acc2 + trb[:, pl.ds(w, 1), :].reshape(72, 128)
        if w >= 10 and (w - 10) % 3 == 0:
            c2t[(w - 10) // 3] = acc2
        if 2 <= w <= 116 and (w - 2) % 3 == 0:
            c2b[(w + 1) // 3] = acc2

    lane_i = jax.lax.broadcasted_iota(jnp.int32, (72, 128), 1)
    dci_sub = jax.lax.broadcasted_iota(jnp.int32, (72, 128), 0)
    rowpos = 3 * lane_i + dr
    for wj in range(40):
        a = c2t[wj]
        b2 = c2b[wj]
        e1 = a - pltpu.roll(a, 88, axis=1)
        e2 = e1 - b2
        box = e2 + pltpu.roll(b2, 88, axis=1)
        ncand = nq_ref[0, 0, 3 * wj:3 * wj + 72, :]
        d = (ncand - 2.0 * box) / 121.0
        colpos = 3 * wj + dci_sub - 32
        valid = ((rowpos >= 0) & (rowpos <= 117) & (colpos >= 0)
                 & (colpos <= 117) & (dci_sub <= 64) & (lane_i < 40))
        d = jnp.where(valid, d, jnp.inf)
        d = jnp.where((dri == 32) & (dci_sub == 32), -jnp.inf, d)
        s = (40 * wj) % 128
        b0 = (40 * wj) // 128
        rolled = pltpu.roll(d, s, axis=1)
        m0 = (lane_i >= s) & (lane_i < min(s + 40, 128))
        stage[:, 128 * b0:128 * b0 + 128] = jnp.where(
            m0, rolled, stage[:, 128 * b0:128 * b0 + 128])
        if s + 40 > 128:
            m1 = lane_i < (s + 40 - 128)
            stage[:, 128 * (b0 + 1):128 * (b0 + 2)] = jnp.where(
                m1, rolled, stage[:, 128 * (b0 + 1):128 * (b0 + 2)])
    d_ref[0, 0] = stage[:, :].reshape(9, 8, NPOSP)


def _topk_kernel(d_ref, gidx_ref, vs, enc):
    pb = pl.program_id(1)
    vs[:, :] = d_ref[0].reshape(NROW, 128)
    enc[:, :] = jax.lax.broadcasted_iota(jnp.int32, (NROW, 128), 0)
    lane = jax.lax.broadcasted_iota(jnp.int32, (1, 128), 1)
    posp = pb * 128 + lane
    wj = posp // 40
    hi = posp - 40 * wj

    def k_body(k, _):
        v = vs[:, :]
        e = enc[:, :]
        m = jnp.min(v, axis=0, keepdims=True)
        ii = jnp.min(jnp.where(v == m, e, jnp.int32(0x3FFFFFFF)),
                     axis=0, keepdims=True)
        vs[:, :] = jnp.where(e == ii, jnp.inf, v)
        dri = ii // 72
        dci = ii - 72 * dri
        row = 3 * hi + dri - 32
        col = 3 * wj + dci - 32
        gidx_ref[0, pl.ds(k, 1), :] = row * 256 + col
        return 0

    jax.lax.fori_loop(0, KSEL, k_body, 0)


def _gather_kernel(x_ref, gidx_ref, y_ref, smem, sem):
    copy = pltpu.make_async_copy(gidx_ref.at[0], smem, sem)
    copy.start()
    copy.wait()
    for li in range(128):
        for k in range(KSEL):
            g = smem[k, li]
            gi = g >> 8
            gj = g & 255
            rows = x_ref[0, pl.ds(gi, P), :]
            rolled = pltpu.roll(rows, (128 - gj) & 127, axis=1)
            y_ref[0, li, k, :, :] = rolled[:, 0:P]


def _box_sum_host(img, p):
    c = jnp.cumsum(jnp.cumsum(img, axis=1), axis=2)
    c = jnp.pad(c, ((0, 0), (1, 0), (1, 0)))
    return c[:, p:, p:] - c[:, :-p, p:] - c[:, p:, :-p] + c[:, :-p, :-p]


@jax.jit
def kernel(input_y):
    x = input_y[:, 0]
    n = x.shape[0]

    xpad = jnp.pad(x, ((0, 0), (V, V), (V, 128 - V)))
    xsh = jnp.stack([xpad[:, :, s:s + 128] for s in range(72)], axis=2)

    norm = _box_sum_host(x * x, P)
    normep = jnp.pad(norm, ((0, 0), (V, V), (V, V)), mode="edge")
    rows = 3 * jnp.arange(NH)[None, :] + jnp.arange(65)[:, None]
    nq = jnp.transpose(normep[:, rows, :], (0, 1, 3, 2))
    nq = jnp.pad(nq, ((0, 0), (0, 0), (0, 10), (0, 88)))

    dists = pl.pallas_call(
        _dist_kernel,
        grid=(n, 65),
        in_specs=[
            pl.BlockSpec((1, 128, 128), lambda i, j: (i, 0, 0)),
            pl.BlockSpec((1, 192, 72, 128), lambda i, j: (i, 0, 0, 0)),
            pl.BlockSpec((1, 1, 192, 128), lambda i, j: (i, j, 0, 0)),
        ],
        out_specs=pl.BlockSpec((1, 1, 9, 8, NPOSP), lambda i, j: (i, j, 0, 0, 0)),
        out_shape=jax.ShapeDtypeStruct((n, 65, 9, 8, NPOSP), jnp.float32),
        scratch_shapes=[
            pltpu.VMEM((72, 80, 128), jnp.float32),
            pltpu.VMEM((72, 128, 128), jnp.float32),
            pltpu.VMEM((40, 72, 128), jnp.float32),
            pltpu.VMEM((40, 72, 128), jnp.float32),
            pltpu.VMEM((72, NPOSP), jnp.float32),
        ],
        compiler_params=pltpu.CompilerParams(
            dimension_semantics=("parallel", "parallel"),
        ),
    )(x, xsh, nq)

    gidx = pl.pallas_call(
        _topk_kernel,
        grid=(n, NPOSP // 128),
        in_specs=[pl.BlockSpec((1, 65, 9, 8, 128), lambda i, j: (i, 0, 0, 0, j))],
        out_specs=pl.BlockSpec((1, KSEL, 128), lambda i, j: (i, 0, j)),
        out_shape=jax.ShapeDtypeStruct((n, KSEL, NPOSP), jnp.int32),
        scratch_shapes=[
            pltpu.VMEM((NROW, 128), jnp.float32),
            pltpu.VMEM((NROW, 128), jnp.int32),
        ],
        compiler_params=pltpu.CompilerParams(
            dimension_semantics=("parallel", "parallel"),
        ),
    )(dists)

    y5 = pl.pallas_call(
        _gather_kernel,
        grid=(n, NPOSP // 128),
        in_specs=[
            pl.BlockSpec((1, 128, 128), lambda i, j: (i, 0, 0)),
            pl.BlockSpec((1, KSEL, 128), lambda i, j: (i, 0, j)),
        ],
        out_specs=pl.BlockSpec((1, 128, KSEL, P, P), lambda i, j: (i, j, 0, 0, 0)),
        out_shape=jax.ShapeDtypeStruct((n, NPOSP, KSEL, P, P), jnp.float32),
        scratch_shapes=[
            pltpu.SMEM((KSEL, 128), jnp.int32),
            pltpu.SemaphoreType.DMA,
        ],
        compiler_params=pltpu.CompilerParams(
            dimension_semantics=("parallel", "parallel"),
        ),
    )(x, gidx)

    y = y5[:, :NPOS].reshape(n, NH, NH, KSEL, P * P)
    return jnp.transpose(y, (0, 2, 1, 3, 4)).reshape(n, NPOS, KSEL, P * P)
